# baseline (speedup 1.0000x reference)
import jax
import jax.numpy as jnp
from jax import lax
from jax.experimental import pallas as pl
from jax.experimental.pallas import tpu as pltpu
from jax.experimental.pallas import tpu_sc as plsc

_NC = 2
_NS = 16
_NW = _NC * _NS
_D = 64
_S = 4096
_T = 200
_LANE = 128
_SUB = 8
_DT = _D // _SUB
_NBUF = 4


def _body(tok_hbm, tab_hbm, out_hbm, idx_v, rows_v, tp_v, gsem, ssem):
    wid = lax.axis_index("s") * _NC + lax.axis_index("c")
    pltpu.sync_copy(tok_hbm.at[:, pl.ds(wid * _LANE, _LANE)], idx_v)

    def gather(j, b):
        return pltpu.make_async_copy(tab_hbm.at[idx_v.at[j]], rows_v.at[b], gsem.at[b])

    def store(j, b):
        return pltpu.make_async_copy(tp_v.at[b], out_hbm.at[j, :, wid], ssem.at[b])

    iota = lax.iota(jnp.int32, 16)
    _c = [(iota + t) & 15 for t in range(16)]
    _n = [((iota + t) & 15) * _LANE + iota for t in range(16)]

    def transpose(b):
        @plsc.parallel_loop(0, (_LANE // 16) * (_D // 16))
        def _(blk):
            ilg = blk >> 2
            dg = blk & 3
            row = ilg * 16 + iota
            vals = [plsc.load_gather(rows_v.at[b], [row, dg * 16 + _c[t]])
                    for t in range(16)]
            for t in range(16):
                a0 = dg * 2 + (_c[t] >> 3)
                a1 = _n[t] + ilg * 16
                plsc.store_scatter(tp_v.at[b], [a0, a1 & 1023], vals[t])

    for b in range(_NBUF):
        gather(b, b).start()

    @pl.loop(0, _T, step=_NBUF)
    def _(g):
        for b in range(_NBUF):
            j = g + b
            gather(j, b).wait()

            @pl.when(g > 0)
            def _():
                store(j - _NBUF, b).wait()

            transpose(b)
            store(j, b).start()

            @pl.when(g < _T - _NBUF)
            def _():
                gather(j + _NBUF, b).start()

    for b in range(_NBUF):
        store(_T - _NBUF + b, b).wait()


@jax.jit
def kernel(token_ids, embeddings):
    tok_t = token_ids.T.astype(jnp.int32)
    out5 = pl.kernel(
        _body,
        out_type=jax.ShapeDtypeStruct((_T, _DT, _NW, _SUB * _LANE), jnp.float32),
        mesh=plsc.VectorSubcoreMesh(core_axis_name="c", subcore_axis_name="s"),
        compiler_params=pltpu.CompilerParams(
            use_tc_tiling_on_sc=False, needs_layout_passes=False),
        scratch_types=[
            pltpu.VMEM((_T, _LANE), jnp.int32),
            pltpu.VMEM((_NBUF, _LANE, _D), jnp.float32),
            pltpu.VMEM((_NBUF, _DT, _SUB * _LANE), jnp.float32),
            pltpu.SemaphoreType.DMA((_NBUF,)),
            pltpu.SemaphoreType.DMA((_NBUF,)),
        ],
    )(tok_t, embeddings)
    out5 = out5.reshape(_T, _DT, _NW, _SUB, _LANE)
    return out5.transpose((2, 4, 0, 1, 3)).reshape(_S, _T, _D)


# speedup vs baseline: 1.4788x; 1.4788x over previous
import jax
import jax.numpy as jnp
from jax import lax
from jax.experimental import pallas as pl
from jax.experimental.pallas import tpu as pltpu
from jax.experimental.pallas import tpu_sc as plsc

_NC = 2
_NS = 16
_NW = _NC * _NS
_D = 64
_S = 4096
_T = 200
_LANE = 128
_SUB = 8
_DT = _D // _SUB
_NBUF = 5


def _body(tok_hbm, tab_hbm, out_hbm, idx_v, rows_v, tp_v, gsem, ssem):
    wid = lax.axis_index("s") * _NC + lax.axis_index("c")
    pltpu.sync_copy(tok_hbm.at[:, pl.ds(wid * _LANE, _LANE)], idx_v)

    def gather(j, b):
        return pltpu.make_async_copy(tab_hbm.at[idx_v.at[j]], rows_v.at[b], gsem.at[b])

    def store(j, b):
        return pltpu.make_async_copy(tp_v.at[b], out_hbm.at[j, :, wid], ssem.at[b])

    iota = lax.iota(jnp.int32, 16)
    _c = [(iota + t) & 15 for t in range(16)]
    _n = [((iota + t) & 15) * _LANE + iota for t in range(16)]

    def transpose(b):
        @plsc.parallel_loop(0, (_LANE // 16) * (_D // 16) * 16, unroll=16)
        def _(q):
            blk = q >> 4
            t = q & 15
            ilg = blk >> 2
            dg = blk & 3
            c = (iota + t) & 15
            row = ilg * 16 + iota
            col = dg * 16 + c
            val = plsc.load_gather(rows_v.at[b], [row, col])
            a1 = (c & 7) * _LANE + row
            plsc.store_scatter(tp_v.at[b], [dg * 2 + (c >> 3), a1], val)

    for b in range(_NBUF):
        gather(b, b).start()

    @pl.loop(0, _T, step=_NBUF)
    def _(g):
        for b in range(_NBUF):
            j = g + b
            gather(j, b).wait()

            @pl.when(g > 0)
            def _():
                store(j - _NBUF, b).wait()

            transpose(b)
            store(j, b).start()

            @pl.when(g < _T - _NBUF)
            def _():
                gather(j + _NBUF, b).start()

    for b in range(_NBUF):
        store(_T - _NBUF + b, b).wait()


@jax.jit
def kernel(token_ids, embeddings):
    tok_t = token_ids.T.astype(jnp.int32)
    out5 = pl.kernel(
        _body,
        out_type=jax.ShapeDtypeStruct((_T, _DT, _NW, _SUB * _LANE), jnp.float32),
        mesh=plsc.VectorSubcoreMesh(core_axis_name="c", subcore_axis_name="s"),
        compiler_params=pltpu.CompilerParams(
            use_tc_tiling_on_sc=False, needs_layout_passes=False),
        scratch_types=[
            pltpu.VMEM((_T, _LANE), jnp.int32),
            pltpu.VMEM((_NBUF, _LANE, _D), jnp.float32),
            pltpu.VMEM((_NBUF, _DT, _SUB * _LANE), jnp.float32),
            pltpu.SemaphoreType.DMA((_NBUF,)),
            pltpu.SemaphoreType.DMA((_NBUF,)),
        ],
    )(tok_t, embeddings)
    out5 = out5.reshape(_T, _DT, _NW, _SUB, _LANE)
    return out5.transpose((2, 4, 0, 1, 3)).reshape(_S, _T, _D)


# speedup vs baseline: 1.6414x; 1.1100x over previous
import jax
import jax.numpy as jnp
from jax import lax
from jax.experimental import pallas as pl
from jax.experimental.pallas import tpu as pltpu
from jax.experimental.pallas import tpu_sc as plsc

_NC = 2
_NS = 16
_NW = _NC * _NS
_D = 64
_S = 4096
_T = 200
_LANE = 128
_SUB = 8
_DT = _D // _SUB
_NBUF = 5


def _body(tok_hbm, tab_hbm, out_hbm, idx_v, rows_v, tp_v, gsem, ssem):
    wid = lax.axis_index("s") * _NC + lax.axis_index("c")
    pltpu.sync_copy(tok_hbm.at[:, pl.ds(wid * _LANE, _LANE)], idx_v)

    def gather(j, b):
        return pltpu.make_async_copy(tab_hbm.at[idx_v.at[j]], rows_v.at[b], gsem.at[b])

    def store(j, b):
        return pltpu.make_async_copy(tp_v.at[b], out_hbm.at[j, :, wid], ssem.at[b])

    iota = lax.iota(jnp.int32, 16)
    _c = [(iota + t) & 15 for t in range(16)]
    _n = [((iota + t) & 15) * _LANE + iota for t in range(16)]

    def transpose(b):
        @plsc.parallel_loop(0, (_LANE // 16) * (_D // 16) * 16, unroll=32)
        def _(q):
            blk = q >> 4
            t = q & 15
            ilg = blk >> 2
            dg = blk & 3
            c = (iota + t) & 15
            row = ilg * 16 + iota
            col = dg * 16 + c
            val = plsc.load_gather(rows_v.at[b], [row, col])
            a1 = (c & 7) * _LANE + row
            plsc.store_scatter(tp_v.at[b], [dg * 2 + (c >> 3), a1], val)

    for b in range(_NBUF):
        gather(b, b).start()

    @pl.loop(0, _T, step=_NBUF)
    def _(g):
        for b in range(_NBUF):
            j = g + b
            gather(j, b).wait()

            @pl.when(g > 0)
            def _():
                store(j - _NBUF, b).wait()

            transpose(b)
            store(j, b).start()

            @pl.when(g < _T - _NBUF)
            def _():
                gather(j + _NBUF, b).start()

    for b in range(_NBUF):
        store(_T - _NBUF + b, b).wait()


@jax.jit
def kernel(token_ids, embeddings):
    tok_t = token_ids.T.astype(jnp.int32)
    out5 = pl.kernel(
        _body,
        out_type=jax.ShapeDtypeStruct((_T, _DT, _NW, _SUB * _LANE), jnp.float32),
        mesh=plsc.VectorSubcoreMesh(core_axis_name="c", subcore_axis_name="s"),
        compiler_params=pltpu.CompilerParams(
            use_tc_tiling_on_sc=False, needs_layout_passes=False),
        scratch_types=[
            pltpu.VMEM((_T, _LANE), jnp.int32),
            pltpu.VMEM((_NBUF, _LANE, _D), jnp.float32),
            pltpu.VMEM((_NBUF, _DT, _SUB * _LANE), jnp.float32),
            pltpu.SemaphoreType.DMA((_NBUF,)),
            pltpu.SemaphoreType.DMA((_NBUF,)),
        ],
    )(tok_t, embeddings)
    out5 = out5.reshape(_T, _DT, _NW, _SUB, _LANE)
    return out5.transpose((2, 4, 0, 1, 3)).reshape(_S, _T, _D)
